# baseline (speedup 1.0000x reference)
import functools

import jax
import jax.numpy as jnp
from jax import lax
from jax.experimental import pallas as pl
from jax.experimental.pallas import tpu as pltpu
from jax.experimental.pallas import tpu_sc as plsc

N = 10000
E = 320000
D = 128
M = 3

NC = 2
NS = 16
NW = NC * NS
CH = 128
EPT = E // NW
NCHUNK = 2 * (-(-EPT // (2 * CH)))
EPT_PAD = NCHUNK * CH
E_PAD = EPT_PAD * NW
NPAD = 10240
DEGW = 16
STRIPE = NPAD // NS

BN = 400
GN = N // BN


def _mm_body(f0, f1, f2, w0, w1, w2, o0, o1, o2):
    o0[...] = jnp.dot(f0[...], w0[...], preferred_element_type=jnp.float32)
    o1[...] = jnp.dot(f1[...], w1[...], preferred_element_type=jnp.float32)
    o2[...] = jnp.dot(f2[...], w2[...], preferred_element_type=jnp.float32)


def _mm(feats, Ws):
    fspec = pl.BlockSpec((BN, D), lambda g: (g, 0))
    wspec = pl.BlockSpec((D, D), lambda g: (0, 0))
    ospec = pl.BlockSpec((BN, D), lambda g: (g, 0))
    return pl.pallas_call(
        _mm_body,
        grid=(GN,),
        in_specs=[fspec, fspec, fspec, wspec, wspec, wspec],
        out_specs=[ospec, ospec, ospec],
        out_shape=[jax.ShapeDtypeStruct((N, D), jnp.float32)] * M,
    )(*feats, *Ws)


def _sc_body(xw0, xw1, xw2, src_hbm, dst_hbm, aggp_hbm, degp_hbm,
             src_a, dst_a, src_b, dst_b, rows_a, rows_b, deg_local,
             agg_s, sem_ga, sem_gb, sem_sa, sem_sb):
    c = lax.axis_index("c")
    s = lax.axis_index("s")
    wid = c * NS + s
    base = s * STRIPE
    ebase = wid * EPT_PAD
    zero16 = jnp.zeros((16,), jnp.float32)
    one16 = jnp.ones((16,), jnp.float32)

    def _one_metapath(m, xw_hbm):
        def _zr(i, _):
            def _zc(j, _):
                rows_a[i, pl.ds(j * 16, 16)] = zero16
                return 0
            return lax.fori_loop(0, D // 16, _zc, 0)
        lax.fori_loop(0, CH, _zr, 0)

        def _zd(i, _):
            deg_local[pl.ds(i * 16, 16)] = zero16
            return 0
        lax.fori_loop(0, NPAD // 16, _zd, 0)

        def _zs(k, _):
            pltpu.sync_copy(rows_a, agg_s.at[pl.ds(base + k * CH, CH)])
            return 0
        lax.fori_loop(0, STRIPE // CH, _zs, 0)

        plsc.subcore_barrier()

        def _deg_acc(dst_v):
            def _deg(q, _):
                idx16 = dst_v[pl.ds(q * 16, 16)]
                plsc.addupdate_scatter(deg_local, [idx16], one16)
                return 0
            lax.fori_loop(0, CH // 16, _deg, 0)

        pltpu.sync_copy(src_hbm.at[m, pl.ds(ebase, CH)], src_a)
        pltpu.sync_copy(dst_hbm.at[m, pl.ds(ebase, CH)], dst_a)
        pltpu.async_copy(xw_hbm.at[src_a], rows_a, sem_ga)

        def _pair(k, _):
            pltpu.make_async_copy(xw_hbm.at[src_a], rows_a, sem_ga).wait()
            eb = ebase + (2 * k + 1) * CH
            pltpu.sync_copy(src_hbm.at[m, pl.ds(eb, CH)], src_b)
            pltpu.sync_copy(dst_hbm.at[m, pl.ds(eb, CH)], dst_b)
            pltpu.async_copy(xw_hbm.at[src_b], rows_b, sem_gb)
            pltpu.sync_copy(rows_a, agg_s.at[dst_a], add=True)
            _deg_acc(dst_a)

            pltpu.make_async_copy(xw_hbm.at[src_b], rows_b, sem_gb).wait()

            @pl.when(k < NCHUNK // 2 - 1)
            def _():
                ea = ebase + (2 * k + 2) * CH
                pltpu.sync_copy(src_hbm.at[m, pl.ds(ea, CH)], src_a)
                pltpu.sync_copy(dst_hbm.at[m, pl.ds(ea, CH)], dst_a)
                pltpu.async_copy(xw_hbm.at[src_a], rows_a, sem_ga)
            pltpu.sync_copy(rows_b, agg_s.at[dst_b], add=True)
            _deg_acc(dst_b)
            return 0
        lax.fori_loop(0, NCHUNK // 2, _pair, 0)

        pltpu.sync_copy(deg_local, degp_hbm.at[m, pl.ds(wid * NPAD, NPAD)])
        plsc.subcore_barrier()

        def _wo(k, _):
            b = base + k * CH
            pltpu.sync_copy(agg_s.at[pl.ds(b, CH)], rows_a)
            pltpu.sync_copy(rows_a, aggp_hbm.at[m, c, pl.ds(b, CH)])
            return 0
        lax.fori_loop(0, STRIPE // CH, _wo, 0)

    for m, xw in enumerate((xw0, xw1, xw2)):
        _one_metapath(m, xw)


@functools.lru_cache(maxsize=1)
def _get_sc_agg():
    return pl.kernel(
        _sc_body,
        out_type=(
            jax.ShapeDtypeStruct((M, NC, NPAD, D), jnp.float32),
            jax.ShapeDtypeStruct((M, NW * NPAD), jnp.float32),
        ),
        mesh=plsc.VectorSubcoreMesh(core_axis_name="c", subcore_axis_name="s"),
        compiler_params=pltpu.CompilerParams(
            use_tc_tiling_on_sc=False, needs_layout_passes=False),
        scratch_types=[
            pltpu.VMEM((CH,), jnp.int32),
            pltpu.VMEM((CH,), jnp.int32),
            pltpu.VMEM((CH,), jnp.int32),
            pltpu.VMEM((CH,), jnp.int32),
            pltpu.VMEM((CH, D), jnp.float32),
            pltpu.VMEM((CH, D), jnp.float32),
            pltpu.VMEM((NPAD,), jnp.float32),
            pltpu.VMEM_SHARED((NPAD, D), jnp.float32),
            pltpu.SemaphoreType.DMA,
            pltpu.SemaphoreType.DMA,
            pltpu.SemaphoreType.DMA,
            pltpu.SemaphoreType.DMA,
        ],
    )


def _pad_edges(edge_index):
    pad = E_PAD - E
    fill = jnp.arange(pad, dtype=jnp.int32)
    src = jnp.concatenate([edge_index[0], fill % N])
    dst = jnp.concatenate([edge_index[1], N + fill % (NPAD - N)])
    return src, dst


def _norm_body(a0, a1, a2, d0, d1, d2, fcwT, fcb, bias, pra,
               h0, h1, h2, ssum):
    g = pl.program_id(0)

    @pl.when(g == 0)
    def _():
        ssum[...] = jnp.zeros((8, D), jnp.float32)

    srows = []
    for m, (ar, dr, ho) in enumerate(((a0, d0, h0), (a1, d1, h1), (a2, d2, h2))):
        av = ar[...]
        agg = av[0] + av[1]
        dv = dr[...]
        deg = jnp.sum(dv, axis=1, keepdims=True)
        deg = jnp.maximum(deg, 1.0)
        h = agg / deg + bias[...][m:m + 1, :]
        a_row = pra[...][m:m + 1, :]
        h = jnp.where(h > 0, h, a_row * h)
        ho[...] = h
        t = jnp.tanh(jnp.dot(h, fcwT[...], preferred_element_type=jnp.float32)
                     + fcb[...][0:1, :])
        srows.append(jnp.sum(t, axis=0, keepdims=True))
    srows.append(jnp.zeros((8 - M, D), jnp.float32))
    ssum[...] += jnp.concatenate(srows, axis=0)


def _norm(aggps, degps, fcwT, fcb_pad, bias_pad, apad):
    aspec = pl.BlockSpec((NC, BN, D), lambda g: (0, g, 0))
    dspec = pl.BlockSpec((BN, NW), lambda g: (g, 0))
    small = pl.BlockSpec((8, D), lambda g: (0, 0))
    wspec = pl.BlockSpec((D, D), lambda g: (0, 0))
    hspec = pl.BlockSpec((BN, D), lambda g: (g, 0))
    return pl.pallas_call(
        _norm_body,
        grid=(GN,),
        in_specs=[aspec, aspec, aspec, dspec, dspec, dspec,
                  wspec, small, small, small],
        out_specs=[hspec, hspec, hspec, small],
        out_shape=[jax.ShapeDtypeStruct((N, D), jnp.float32)] * M
        + [jax.ShapeDtypeStruct((8, D), jnp.float32)],
    )(*aggps, *degps, fcwT, fcb_pad, bias_pad, apad)


def _comb_body(h0, h1, h2, ssum, attnp, out):
    sv = ssum[...]
    prod = sv * attnp[...][0:1, :]
    w = jnp.sum(prod, axis=1, keepdims=True) * (1.0 / N)
    rid = lax.broadcasted_iota(jnp.int32, (8, 1), 0)
    valid = rid < M
    wm = jnp.where(valid, w, -1e30)
    mx = jnp.max(wm, axis=0, keepdims=True)
    ex = jnp.where(valid, jnp.exp(wm - mx), 0.0)
    beta = ex / jnp.sum(ex, axis=0, keepdims=True)
    out[...] = (beta[0:1] * h0[...] + beta[1:2] * h1[...] + beta[2:3] * h2[...])


def _comb(h, ssum, attn_pad):
    hspec = pl.BlockSpec((BN, D), lambda g: (g, 0))
    small = pl.BlockSpec((8, D), lambda g: (0, 0))
    return pl.pallas_call(
        _comb_body,
        grid=(GN,),
        in_specs=[hspec, hspec, hspec, small, small],
        out_specs=hspec,
        out_shape=jax.ShapeDtypeStruct((N, D), jnp.float32),
    )(*h, ssum, attn_pad)


def kernel(feat0, feat1, feat2, edge_index0, edge_index1, edge_index2,
           W0, W1, W2, b0, b1, b2, prelu_a0, prelu_a1, prelu_a2,
           fc_w, fc_b, attn):
    xws = _mm((feat0, feat1, feat2), (W0, W1, W2))

    pads = [_pad_edges(ei) for ei in (edge_index0, edge_index1, edge_index2)]
    src_all = jnp.stack([p[0] for p in pads])
    dst_all = jnp.stack([p[1] for p in pads])
    aggp, degp = _get_sc_agg()(*xws, src_all, dst_all)
    aggps = [aggp[m] for m in range(M)]
    degps = [degp[m].reshape(NW, NPAD)[:, :N].transpose(1, 0)
             for m in range(M)]

    zrow = jnp.zeros((8, D), jnp.float32)
    bias_pad = zrow.at[0].set(b0).at[1].set(b1).at[2].set(b2)
    apad = zrow.at[0].set(prelu_a0).at[1].set(prelu_a1).at[2].set(prelu_a2)
    fcb_pad = zrow.at[0].set(fc_b)
    attn_pad = zrow.at[0].set(attn[0])

    h0, h1, h2, ssum = _norm(aggps, degps, fc_w.T, fcb_pad, bias_pad, apad)
    return _comb((h0, h1, h2), ssum, attn_pad)


# speedup vs baseline: 1.0175x; 1.0175x over previous
import functools

import jax
import jax.numpy as jnp
from jax import lax
from jax.experimental import pallas as pl
from jax.experimental.pallas import tpu as pltpu
from jax.experimental.pallas import tpu_sc as plsc

N = 10000
E = 320000
D = 128
M = 3

NC = 2
NS = 16
NW = NC * NS
CH = 128
EPT = E // NW
NCHUNK = 2 * (-(-EPT // (2 * CH)))
EPT_PAD = NCHUNK * CH
E_PAD = EPT_PAD * NW
NPAD = 10240
DEGW = 16
STRIPE = NPAD // NS

BN = 400
GN = N // BN


def _mm_body(f0, f1, f2, w0, w1, w2, o0, o1, o2):
    o0[...] = jnp.dot(f0[...], w0[...], preferred_element_type=jnp.float32)
    o1[...] = jnp.dot(f1[...], w1[...], preferred_element_type=jnp.float32)
    o2[...] = jnp.dot(f2[...], w2[...], preferred_element_type=jnp.float32)


def _mm(feats, Ws):
    fspec = pl.BlockSpec((BN, D), lambda g: (g, 0))
    wspec = pl.BlockSpec((D, D), lambda g: (0, 0))
    ospec = pl.BlockSpec((BN, D), lambda g: (g, 0))
    return pl.pallas_call(
        _mm_body,
        grid=(GN,),
        in_specs=[fspec, fspec, fspec, wspec, wspec, wspec],
        out_specs=[ospec, ospec, ospec],
        out_shape=[jax.ShapeDtypeStruct((N, D), jnp.float32)] * M,
    )(*feats, *Ws)


def _sc_body(xw0, xw1, xw2, se0, se1, se2, de0, de1, de2, aggp_hbm, degp_hbm,
             src_a, dst_a, src_b, dst_b, rows_a, rows_b, deg_local,
             agg_s, sem_ga, sem_gb, sem_sa, sem_sb):
    c = lax.axis_index("c")
    s = lax.axis_index("s")
    wid = c * NS + s
    base = s * STRIPE
    ebase = wid * EPT_PAD
    zero16 = jnp.zeros((16,), jnp.float32)
    one16 = jnp.ones((16,), jnp.float32)

    def _one_metapath(m, xw_hbm, src_hbm, dst_hbm):
        def _zr(i, _):
            def _zc(j, _):
                rows_a[i, pl.ds(j * 16, 16)] = zero16
                return 0
            return lax.fori_loop(0, D // 16, _zc, 0)
        lax.fori_loop(0, CH, _zr, 0)

        def _zd(i, _):
            deg_local[pl.ds(i * 16, 16)] = zero16
            return 0
        lax.fori_loop(0, NPAD // 16, _zd, 0)

        def _zs(k, _):
            pltpu.sync_copy(rows_a, agg_s.at[pl.ds(base + k * CH, CH)])
            return 0
        lax.fori_loop(0, STRIPE // CH, _zs, 0)

        plsc.subcore_barrier()

        def _deg_acc(dst_v):
            def _deg(q, _):
                idx16 = dst_v[pl.ds(q * 16, 16)]
                plsc.addupdate_scatter(deg_local, [idx16], one16)
                return 0
            lax.fori_loop(0, CH // 16, _deg, 0)

        pltpu.sync_copy(src_hbm.at[pl.ds(ebase, CH)], src_a)
        pltpu.sync_copy(dst_hbm.at[pl.ds(ebase, CH)], dst_a)
        pltpu.async_copy(xw_hbm.at[src_a], rows_a, sem_ga)

        def _pair(k, _):
            pltpu.make_async_copy(xw_hbm.at[src_a], rows_a, sem_ga).wait()
            eb = ebase + (2 * k + 1) * CH
            pltpu.sync_copy(src_hbm.at[pl.ds(eb, CH)], src_b)
            pltpu.sync_copy(dst_hbm.at[pl.ds(eb, CH)], dst_b)
            pltpu.async_copy(xw_hbm.at[src_b], rows_b, sem_gb)
            pltpu.sync_copy(rows_a, agg_s.at[dst_a], add=True)
            _deg_acc(dst_a)

            pltpu.make_async_copy(xw_hbm.at[src_b], rows_b, sem_gb).wait()

            @pl.when(k < NCHUNK // 2 - 1)
            def _():
                ea = ebase + (2 * k + 2) * CH
                pltpu.sync_copy(src_hbm.at[pl.ds(ea, CH)], src_a)
                pltpu.sync_copy(dst_hbm.at[pl.ds(ea, CH)], dst_a)
                pltpu.async_copy(xw_hbm.at[src_a], rows_a, sem_ga)
            pltpu.sync_copy(rows_b, agg_s.at[dst_b], add=True)
            _deg_acc(dst_b)
            return 0
        lax.fori_loop(0, NCHUNK // 2, _pair, 0)

        pltpu.sync_copy(deg_local, degp_hbm.at[m, pl.ds(wid * NPAD, NPAD)])
        plsc.subcore_barrier()

        def _wo(k, _):
            b = base + k * CH
            pltpu.sync_copy(agg_s.at[pl.ds(b, CH)], rows_a)
            pltpu.sync_copy(rows_a, aggp_hbm.at[m, c, pl.ds(b, CH)])
            return 0
        lax.fori_loop(0, STRIPE // CH, _wo, 0)

    for m, (xw, sh, dh) in enumerate(((xw0, se0, de0), (xw1, se1, de1),
                                      (xw2, se2, de2))):
        _one_metapath(m, xw, sh, dh)


@functools.lru_cache(maxsize=1)
def _get_sc_agg():
    return pl.kernel(
        _sc_body,
        out_type=(
            jax.ShapeDtypeStruct((M, NC, NPAD, D), jnp.float32),
            jax.ShapeDtypeStruct((M, NW * NPAD), jnp.float32),
        ),
        mesh=plsc.VectorSubcoreMesh(core_axis_name="c", subcore_axis_name="s"),
        compiler_params=pltpu.CompilerParams(
            use_tc_tiling_on_sc=False, needs_layout_passes=False),
        scratch_types=[
            pltpu.VMEM((CH,), jnp.int32),
            pltpu.VMEM((CH,), jnp.int32),
            pltpu.VMEM((CH,), jnp.int32),
            pltpu.VMEM((CH,), jnp.int32),
            pltpu.VMEM((CH, D), jnp.float32),
            pltpu.VMEM((CH, D), jnp.float32),
            pltpu.VMEM((NPAD,), jnp.float32),
            pltpu.VMEM_SHARED((NPAD, D), jnp.float32),
            pltpu.SemaphoreType.DMA,
            pltpu.SemaphoreType.DMA,
            pltpu.SemaphoreType.DMA,
            pltpu.SemaphoreType.DMA,
        ],
    )


def _pad_edges(edge_index):
    pad = E_PAD - E
    fill = jnp.arange(pad, dtype=jnp.int32)
    src = jnp.concatenate([edge_index[0], fill % N])
    dst = jnp.concatenate([edge_index[1], N + fill % (NPAD - N)])
    return src, dst


def _norm_body(a0, a1, a2, d0, d1, d2, fcwT, fcb, bias, pra,
               h0, h1, h2, ssum):
    g = pl.program_id(0)

    @pl.when(g == 0)
    def _():
        ssum[...] = jnp.zeros((8, D), jnp.float32)

    srows = []
    for m, (ar, dr, ho) in enumerate(((a0, d0, h0), (a1, d1, h1), (a2, d2, h2))):
        av = ar[...]
        agg = av[0] + av[1]
        dv = dr[...]
        deg = jnp.sum(dv, axis=1, keepdims=True)
        deg = jnp.maximum(deg, 1.0)
        h = agg / deg + bias[...][m:m + 1, :]
        a_row = pra[...][m:m + 1, :]
        h = jnp.where(h > 0, h, a_row * h)
        ho[...] = h
        t = jnp.tanh(jnp.dot(h, fcwT[...], preferred_element_type=jnp.float32)
                     + fcb[...][0:1, :])
        srows.append(jnp.sum(t, axis=0, keepdims=True))
    srows.append(jnp.zeros((8 - M, D), jnp.float32))
    ssum[...] += jnp.concatenate(srows, axis=0)


def _norm(aggps, degps, fcwT, fcb_pad, bias_pad, apad):
    aspec = pl.BlockSpec((NC, BN, D), lambda g: (0, g, 0))
    dspec = pl.BlockSpec((BN, NW), lambda g: (g, 0))
    small = pl.BlockSpec((8, D), lambda g: (0, 0))
    wspec = pl.BlockSpec((D, D), lambda g: (0, 0))
    hspec = pl.BlockSpec((BN, D), lambda g: (g, 0))
    return pl.pallas_call(
        _norm_body,
        grid=(GN,),
        in_specs=[aspec, aspec, aspec, dspec, dspec, dspec,
                  wspec, small, small, small],
        out_specs=[hspec, hspec, hspec, small],
        out_shape=[jax.ShapeDtypeStruct((N, D), jnp.float32)] * M
        + [jax.ShapeDtypeStruct((8, D), jnp.float32)],
    )(*aggps, *degps, fcwT, fcb_pad, bias_pad, apad)


def _comb_body(h0, h1, h2, ssum, attnp, out):
    sv = ssum[...]
    prod = sv * attnp[...][0:1, :]
    w = jnp.sum(prod, axis=1, keepdims=True) * (1.0 / N)
    rid = lax.broadcasted_iota(jnp.int32, (8, 1), 0)
    valid = rid < M
    wm = jnp.where(valid, w, -1e30)
    mx = jnp.max(wm, axis=0, keepdims=True)
    ex = jnp.where(valid, jnp.exp(wm - mx), 0.0)
    beta = ex / jnp.sum(ex, axis=0, keepdims=True)
    out[...] = (beta[0:1] * h0[...] + beta[1:2] * h1[...] + beta[2:3] * h2[...])


def _comb(h, ssum, attn_pad):
    hspec = pl.BlockSpec((BN, D), lambda g: (g, 0))
    small = pl.BlockSpec((8, D), lambda g: (0, 0))
    return pl.pallas_call(
        _comb_body,
        grid=(GN,),
        in_specs=[hspec, hspec, hspec, small, small],
        out_specs=hspec,
        out_shape=jax.ShapeDtypeStruct((N, D), jnp.float32),
    )(*h, ssum, attn_pad)


def kernel(feat0, feat1, feat2, edge_index0, edge_index1, edge_index2,
           W0, W1, W2, b0, b1, b2, prelu_a0, prelu_a1, prelu_a2,
           fc_w, fc_b, attn):
    xws = _mm((feat0, feat1, feat2), (W0, W1, W2))

    pads = [_pad_edges(ei) for ei in (edge_index0, edge_index1, edge_index2)]
    aggp, degp = _get_sc_agg()(*xws, pads[0][0], pads[1][0], pads[2][0],
                               pads[0][1], pads[1][1], pads[2][1])
    aggps = [aggp[m] for m in range(M)]
    degps = [degp[m].reshape(NW, NPAD)[:, :N].transpose(1, 0)
             for m in range(M)]

    zrow = jnp.zeros((8, D), jnp.float32)
    bias_pad = zrow.at[0].set(b0).at[1].set(b1).at[2].set(b2)
    apad = zrow.at[0].set(prelu_a0).at[1].set(prelu_a1).at[2].set(prelu_a2)
    fcb_pad = zrow.at[0].set(fc_b)
    attn_pad = zrow.at[0].set(attn[0])

    h0, h1, h2, ssum = _norm(aggps, degps, fc_w.T, fcb_pad, bias_pad, apad)
    return _comb((h0, h1, h2), ssum, attn_pad)


# speedup vs baseline: 1.0718x; 1.0534x over previous
import functools

import jax
import jax.numpy as jnp
from jax import lax
from jax.experimental import pallas as pl
from jax.experimental.pallas import tpu as pltpu
from jax.experimental.pallas import tpu_sc as plsc

N = 10000
E = 320000
D = 128
M = 3

NC = 2
NS = 16
NW = NC * NS
CH = 128
EPT = E // NW
NCHUNK = 2 * (-(-EPT // (2 * CH)))
EPT_PAD = NCHUNK * CH
E_PAD = EPT_PAD * NW
NPAD = 10240
DEGW = 16
STRIPE = NPAD // NS

BN = 400
GN = N // BN


def _mm_body(f0, f1, f2, w0, w1, w2, o0, o1, o2):
    o0[...] = jnp.dot(f0[...], w0[...], preferred_element_type=jnp.float32)
    o1[...] = jnp.dot(f1[...], w1[...], preferred_element_type=jnp.float32)
    o2[...] = jnp.dot(f2[...], w2[...], preferred_element_type=jnp.float32)


def _mm(feats, Ws):
    fspec = pl.BlockSpec((BN, D), lambda g: (g, 0))
    wspec = pl.BlockSpec((D, D), lambda g: (0, 0))
    ospec = pl.BlockSpec((BN, D), lambda g: (g, 0))
    return pl.pallas_call(
        _mm_body,
        grid=(GN,),
        in_specs=[fspec, fspec, fspec, wspec, wspec, wspec],
        out_specs=[ospec, ospec, ospec],
        out_shape=[jax.ShapeDtypeStruct((N, D), jnp.float32)] * M,
    )(*feats, *Ws)


def _sc_body(xw_hbm, src_hbm, dst_hbm, aggp_hbm, degp_hbm,
             src_a, dst_a, src_b, dst_b, rows_a, rows_b, deg_local,
             agg_s, sem_ga, sem_gb, sem_sa, sem_sb):
    c = lax.axis_index("c")
    s = lax.axis_index("s")
    wid = c * NS + s
    base = s * STRIPE
    ebase = wid * EPT_PAD
    zero16 = jnp.zeros((16,), jnp.float32)
    one16 = jnp.ones((16,), jnp.float32)

    if True:
        def _zr(i, _):
            def _zc(j, _):
                rows_a[i, pl.ds(j * 16, 16)] = zero16
                return 0
            return lax.fori_loop(0, D // 16, _zc, 0)
        lax.fori_loop(0, CH, _zr, 0)

        def _zd(i, _):
            deg_local[pl.ds(i * 16, 16)] = zero16
            return 0
        lax.fori_loop(0, NPAD // 16, _zd, 0)

        def _zs(k, _):
            pltpu.sync_copy(rows_a, agg_s.at[pl.ds(base + k * CH, CH)])
            return 0
        lax.fori_loop(0, STRIPE // CH, _zs, 0)

        plsc.subcore_barrier()

        def _deg_acc(dst_v):
            def _deg(q, _):
                idx16 = dst_v[pl.ds(q * 16, 16)]
                plsc.addupdate_scatter(deg_local, [idx16], one16)
                return 0
            lax.fori_loop(0, CH // 16, _deg, 0)

        pltpu.sync_copy(src_hbm.at[pl.ds(ebase, CH)], src_a)
        pltpu.sync_copy(dst_hbm.at[pl.ds(ebase, CH)], dst_a)
        pltpu.async_copy(xw_hbm.at[src_a], rows_a, sem_ga)

        def _pair(k, _):
            pltpu.make_async_copy(xw_hbm.at[src_a], rows_a, sem_ga).wait()
            eb = ebase + (2 * k + 1) * CH
            pltpu.sync_copy(src_hbm.at[pl.ds(eb, CH)], src_b)
            pltpu.sync_copy(dst_hbm.at[pl.ds(eb, CH)], dst_b)
            pltpu.async_copy(xw_hbm.at[src_b], rows_b, sem_gb)
            pltpu.sync_copy(rows_a, agg_s.at[dst_a], add=True)
            _deg_acc(dst_a)

            pltpu.make_async_copy(xw_hbm.at[src_b], rows_b, sem_gb).wait()

            @pl.when(k < NCHUNK // 2 - 1)
            def _():
                ea = ebase + (2 * k + 2) * CH
                pltpu.sync_copy(src_hbm.at[pl.ds(ea, CH)], src_a)
                pltpu.sync_copy(dst_hbm.at[pl.ds(ea, CH)], dst_a)
                pltpu.async_copy(xw_hbm.at[src_a], rows_a, sem_ga)
            pltpu.sync_copy(rows_b, agg_s.at[dst_b], add=True)
            _deg_acc(dst_b)
            return 0
        lax.fori_loop(0, NCHUNK // 2, _pair, 0)

        pltpu.sync_copy(deg_local, degp_hbm.at[pl.ds(wid * NPAD, NPAD)])
        plsc.subcore_barrier()

        def _wo(k, _):
            b = base + k * CH
            pltpu.sync_copy(agg_s.at[pl.ds(b, CH)], rows_a)
            pltpu.sync_copy(rows_a, aggp_hbm.at[c, pl.ds(b, CH)])
            return 0
        lax.fori_loop(0, STRIPE // CH, _wo, 0)


@functools.lru_cache(maxsize=1)
def _get_sc_agg():
    return pl.kernel(
        _sc_body,
        out_type=(
            jax.ShapeDtypeStruct((NC, NPAD, D), jnp.float32),
            jax.ShapeDtypeStruct((NW * NPAD,), jnp.float32),
        ),
        mesh=plsc.VectorSubcoreMesh(core_axis_name="c", subcore_axis_name="s"),
        compiler_params=pltpu.CompilerParams(
            use_tc_tiling_on_sc=False, needs_layout_passes=False),
        scratch_types=[
            pltpu.VMEM((CH,), jnp.int32),
            pltpu.VMEM((CH,), jnp.int32),
            pltpu.VMEM((CH,), jnp.int32),
            pltpu.VMEM((CH,), jnp.int32),
            pltpu.VMEM((CH, D), jnp.float32),
            pltpu.VMEM((CH, D), jnp.float32),
            pltpu.VMEM((NPAD,), jnp.float32),
            pltpu.VMEM_SHARED((NPAD, D), jnp.float32),
            pltpu.SemaphoreType.DMA,
            pltpu.SemaphoreType.DMA,
            pltpu.SemaphoreType.DMA,
            pltpu.SemaphoreType.DMA,
        ],
    )


def _pad_edges(edge_index):
    pad = E_PAD - E
    fill = jnp.arange(pad, dtype=jnp.int32)
    src = jnp.concatenate([edge_index[0], fill % N])
    dst = jnp.concatenate([edge_index[1], N + fill % (NPAD - N)])
    return src, dst


def _norm_body(a0, a1, a2, d0, d1, d2, fcwT, fcb, bias, pra,
               h0, h1, h2, ssum):
    g = pl.program_id(0)

    @pl.when(g == 0)
    def _():
        ssum[...] = jnp.zeros((8, D), jnp.float32)

    srows = []
    for m, (ar, dr, ho) in enumerate(((a0, d0, h0), (a1, d1, h1), (a2, d2, h2))):
        av = ar[...]
        agg = av[0] + av[1]
        dv = dr[...]
        deg = jnp.sum(dv, axis=1, keepdims=True)
        deg = jnp.maximum(deg, 1.0)
        h = agg / deg + bias[...][m:m + 1, :]
        a_row = pra[...][m:m + 1, :]
        h = jnp.where(h > 0, h, a_row * h)
        ho[...] = h
        t = jnp.tanh(jnp.dot(h, fcwT[...], preferred_element_type=jnp.float32)
                     + fcb[...][0:1, :])
        srows.append(jnp.sum(t, axis=0, keepdims=True))
    srows.append(jnp.zeros((8 - M, D), jnp.float32))
    ssum[...] += jnp.concatenate(srows, axis=0)


def _norm(aggps, degps, fcwT, fcb_pad, bias_pad, apad):
    aspec = pl.BlockSpec((NC, BN, D), lambda g: (0, g, 0))
    dspec = pl.BlockSpec((BN, NW), lambda g: (g, 0))
    small = pl.BlockSpec((8, D), lambda g: (0, 0))
    wspec = pl.BlockSpec((D, D), lambda g: (0, 0))
    hspec = pl.BlockSpec((BN, D), lambda g: (g, 0))
    return pl.pallas_call(
        _norm_body,
        grid=(GN,),
        in_specs=[aspec, aspec, aspec, dspec, dspec, dspec,
                  wspec, small, small, small],
        out_specs=[hspec, hspec, hspec, small],
        out_shape=[jax.ShapeDtypeStruct((N, D), jnp.float32)] * M
        + [jax.ShapeDtypeStruct((8, D), jnp.float32)],
    )(*aggps, *degps, fcwT, fcb_pad, bias_pad, apad)


def _comb_body(h0, h1, h2, ssum, attnp, out):
    sv = ssum[...]
    prod = sv * attnp[...][0:1, :]
    w = jnp.sum(prod, axis=1, keepdims=True) * (1.0 / N)
    rid = lax.broadcasted_iota(jnp.int32, (8, 1), 0)
    valid = rid < M
    wm = jnp.where(valid, w, -1e30)
    mx = jnp.max(wm, axis=0, keepdims=True)
    ex = jnp.where(valid, jnp.exp(wm - mx), 0.0)
    beta = ex / jnp.sum(ex, axis=0, keepdims=True)
    out[...] = (beta[0:1] * h0[...] + beta[1:2] * h1[...] + beta[2:3] * h2[...])


def _comb(h, ssum, attn_pad):
    hspec = pl.BlockSpec((BN, D), lambda g: (g, 0))
    small = pl.BlockSpec((8, D), lambda g: (0, 0))
    return pl.pallas_call(
        _comb_body,
        grid=(GN,),
        in_specs=[hspec, hspec, hspec, small, small],
        out_specs=hspec,
        out_shape=jax.ShapeDtypeStruct((N, D), jnp.float32),
    )(*h, ssum, attn_pad)


def kernel(feat0, feat1, feat2, edge_index0, edge_index1, edge_index2,
           W0, W1, W2, b0, b1, b2, prelu_a0, prelu_a1, prelu_a2,
           fc_w, fc_b, attn):
    xws = _mm((feat0, feat1, feat2), (W0, W1, W2))

    aggps, degps = [], []
    for xw, ei in zip(xws, (edge_index0, edge_index1, edge_index2)):
        srcp, dstp = _pad_edges(ei)
        aggp, degp = _get_sc_agg()(xw, srcp, dstp)
        aggps.append(aggp)
        degps.append(degp.reshape(NW, NPAD)[:, :N].transpose(1, 0))

    zrow = jnp.zeros((8, D), jnp.float32)
    bias_pad = zrow.at[0].set(b0).at[1].set(b1).at[2].set(b2)
    apad = zrow.at[0].set(prelu_a0).at[1].set(prelu_a1).at[2].set(prelu_a2)
    fcb_pad = zrow.at[0].set(fc_b)
    attn_pad = zrow.at[0].set(attn[0])

    h0, h1, h2, ssum = _norm(aggps, degps, fc_w.T, fcb_pad, bias_pad, apad)
    return _comb((h0, h1, h2), ssum, attn_pad)


# speedup vs baseline: 1.0722x; 1.0004x over previous
import functools

import jax
import jax.numpy as jnp
from jax import lax
from jax.experimental import pallas as pl
from jax.experimental.pallas import tpu as pltpu
from jax.experimental.pallas import tpu_sc as plsc

N = 10000
E = 320000
D = 128
M = 3

NC = 2
NS = 16
NW = NC * NS
CH = 128
EPT = E // NW
NCHUNK = 2 * (-(-EPT // (2 * CH)))
EPT_PAD = NCHUNK * CH
E_PAD = EPT_PAD * NW
NPAD = 10240
DEGW = 16
STRIPE = NPAD // NS

BN = 400
GN = N // BN


def _mm_body(f0, f1, f2, w0, w1, w2, o0, o1, o2):
    o0[...] = jnp.dot(f0[...], w0[...], preferred_element_type=jnp.float32)
    o1[...] = jnp.dot(f1[...], w1[...], preferred_element_type=jnp.float32)
    o2[...] = jnp.dot(f2[...], w2[...], preferred_element_type=jnp.float32)


def _mm(feats, Ws):
    fspec = pl.BlockSpec((BN, D), lambda g: (g, 0))
    wspec = pl.BlockSpec((D, D), lambda g: (0, 0))
    ospec = pl.BlockSpec((BN, D), lambda g: (g, 0))
    return pl.pallas_call(
        _mm_body,
        grid=(GN,),
        in_specs=[fspec, fspec, fspec, wspec, wspec, wspec],
        out_specs=[ospec, ospec, ospec],
        out_shape=[jax.ShapeDtypeStruct((N, D), jnp.float32)] * M,
    )(*feats, *Ws)


def _sc_body(xw_hbm, src_hbm, dst_hbm, aggp_hbm, degp_hbm,
             src_a, dst_a, src_b, dst_b, rows_a, rows_b, deg_local,
             agg_s, sem_ga, sem_gb, sem_sa, sem_sb):
    c = lax.axis_index("c")
    s = lax.axis_index("s")
    wid = c * NS + s
    base = s * STRIPE
    ebase = wid * EPT_PAD
    zero16 = jnp.zeros((16,), jnp.float32)
    one16 = jnp.ones((16,), jnp.float32)

    if True:
        def _zr(i, _):
            def _zc(j, _):
                rows_a[i, pl.ds(j * 16, 16)] = zero16
                return 0
            return lax.fori_loop(0, D // 16, _zc, 0)
        lax.fori_loop(0, CH, _zr, 0)

        def _zd(i, _):
            deg_local[pl.ds(i * 16, 16)] = zero16
            return 0
        lax.fori_loop(0, NPAD // 16, _zd, 0)

        def _zs(k, _):
            pltpu.sync_copy(rows_a, agg_s.at[pl.ds(base + k * CH, CH)])
            return 0
        lax.fori_loop(0, STRIPE // CH, _zs, 0)

        plsc.subcore_barrier()

        def _deg_acc(dst_v):
            def _deg(q, _):
                idx16 = dst_v[pl.ds(q * 16, 16)]
                plsc.addupdate_scatter(deg_local, [idx16], one16)
                return 0
            lax.fori_loop(0, CH // 16, _deg, 0)

        pltpu.sync_copy(src_hbm.at[pl.ds(ebase, CH)], src_a)
        pltpu.sync_copy(dst_hbm.at[pl.ds(ebase, CH)], dst_a)
        pltpu.async_copy(xw_hbm.at[src_a], rows_a, sem_ga)

        def _pair(k, _):
            pltpu.make_async_copy(xw_hbm.at[src_a], rows_a, sem_ga).wait()
            eb = ebase + (2 * k + 1) * CH
            pltpu.sync_copy(src_hbm.at[pl.ds(eb, CH)], src_b)
            pltpu.sync_copy(dst_hbm.at[pl.ds(eb, CH)], dst_b)
            pltpu.async_copy(xw_hbm.at[src_b], rows_b, sem_gb)
            pltpu.sync_copy(rows_a, agg_s.at[dst_a], add=True)
            _deg_acc(dst_a)

            pltpu.make_async_copy(xw_hbm.at[src_b], rows_b, sem_gb).wait()

            @pl.when(k < NCHUNK // 2 - 1)
            def _():
                ea = ebase + (2 * k + 2) * CH
                pltpu.sync_copy(src_hbm.at[pl.ds(ea, CH)], src_a)
                pltpu.sync_copy(dst_hbm.at[pl.ds(ea, CH)], dst_a)
                pltpu.async_copy(xw_hbm.at[src_a], rows_a, sem_ga)
            pltpu.sync_copy(rows_b, agg_s.at[dst_b], add=True)
            _deg_acc(dst_b)
            return 0
        lax.fori_loop(0, NCHUNK // 2, _pair, 0)

        pltpu.sync_copy(deg_local, degp_hbm.at[pl.ds(wid * NPAD, NPAD)])
        plsc.subcore_barrier()

        pltpu.sync_copy(agg_s.at[pl.ds(base, STRIPE)],
                        aggp_hbm.at[c, pl.ds(base, STRIPE)])


@functools.lru_cache(maxsize=1)
def _get_sc_agg():
    return pl.kernel(
        _sc_body,
        out_type=(
            jax.ShapeDtypeStruct((NC, NPAD, D), jnp.float32),
            jax.ShapeDtypeStruct((NW * NPAD,), jnp.float32),
        ),
        mesh=plsc.VectorSubcoreMesh(core_axis_name="c", subcore_axis_name="s"),
        compiler_params=pltpu.CompilerParams(
            use_tc_tiling_on_sc=False, needs_layout_passes=False),
        scratch_types=[
            pltpu.VMEM((CH,), jnp.int32),
            pltpu.VMEM((CH,), jnp.int32),
            pltpu.VMEM((CH,), jnp.int32),
            pltpu.VMEM((CH,), jnp.int32),
            pltpu.VMEM((CH, D), jnp.float32),
            pltpu.VMEM((CH, D), jnp.float32),
            pltpu.VMEM((NPAD,), jnp.float32),
            pltpu.VMEM_SHARED((NPAD, D), jnp.float32),
            pltpu.SemaphoreType.DMA,
            pltpu.SemaphoreType.DMA,
            pltpu.SemaphoreType.DMA,
            pltpu.SemaphoreType.DMA,
        ],
    )


def _pad_edges(edge_index):
    pad = E_PAD - E
    fill = jnp.arange(pad, dtype=jnp.int32)
    src = jnp.concatenate([edge_index[0], fill % N])
    dst = jnp.concatenate([edge_index[1], N + fill % (NPAD - N)])
    return src, dst


def _norm_body(a0, a1, a2, d0, d1, d2, fcwT, fcb, bias, pra,
               h0, h1, h2, ssum):
    g = pl.program_id(0)

    @pl.when(g == 0)
    def _():
        ssum[...] = jnp.zeros((8, D), jnp.float32)

    srows = []
    for m, (ar, dr, ho) in enumerate(((a0, d0, h0), (a1, d1, h1), (a2, d2, h2))):
        av = ar[...]
        agg = av[0] + av[1]
        dv = dr[...]
        deg = jnp.sum(dv, axis=1, keepdims=True)
        deg = jnp.maximum(deg, 1.0)
        h = agg / deg + bias[...][m:m + 1, :]
        a_row = pra[...][m:m + 1, :]
        h = jnp.where(h > 0, h, a_row * h)
        ho[...] = h
        t = jnp.tanh(jnp.dot(h, fcwT[...], preferred_element_type=jnp.float32)
                     + fcb[...][0:1, :])
        srows.append(jnp.sum(t, axis=0, keepdims=True))
    srows.append(jnp.zeros((8 - M, D), jnp.float32))
    ssum[...] += jnp.concatenate(srows, axis=0)


def _norm(aggps, degps, fcwT, fcb_pad, bias_pad, apad):
    aspec = pl.BlockSpec((NC, BN, D), lambda g: (0, g, 0))
    dspec = pl.BlockSpec((BN, NW), lambda g: (g, 0))
    small = pl.BlockSpec((8, D), lambda g: (0, 0))
    wspec = pl.BlockSpec((D, D), lambda g: (0, 0))
    hspec = pl.BlockSpec((BN, D), lambda g: (g, 0))
    return pl.pallas_call(
        _norm_body,
        grid=(GN,),
        in_specs=[aspec, aspec, aspec, dspec, dspec, dspec,
                  wspec, small, small, small],
        out_specs=[hspec, hspec, hspec, small],
        out_shape=[jax.ShapeDtypeStruct((N, D), jnp.float32)] * M
        + [jax.ShapeDtypeStruct((8, D), jnp.float32)],
    )(*aggps, *degps, fcwT, fcb_pad, bias_pad, apad)


def _comb_body(h0, h1, h2, ssum, attnp, out):
    sv = ssum[...]
    prod = sv * attnp[...][0:1, :]
    w = jnp.sum(prod, axis=1, keepdims=True) * (1.0 / N)
    rid = lax.broadcasted_iota(jnp.int32, (8, 1), 0)
    valid = rid < M
    wm = jnp.where(valid, w, -1e30)
    mx = jnp.max(wm, axis=0, keepdims=True)
    ex = jnp.where(valid, jnp.exp(wm - mx), 0.0)
    beta = ex / jnp.sum(ex, axis=0, keepdims=True)
    out[...] = (beta[0:1] * h0[...] + beta[1:2] * h1[...] + beta[2:3] * h2[...])


def _comb(h, ssum, attn_pad):
    hspec = pl.BlockSpec((BN, D), lambda g: (g, 0))
    small = pl.BlockSpec((8, D), lambda g: (0, 0))
    return pl.pallas_call(
        _comb_body,
        grid=(GN,),
        in_specs=[hspec, hspec, hspec, small, small],
        out_specs=hspec,
        out_shape=jax.ShapeDtypeStruct((N, D), jnp.float32),
    )(*h, ssum, attn_pad)


def kernel(feat0, feat1, feat2, edge_index0, edge_index1, edge_index2,
           W0, W1, W2, b0, b1, b2, prelu_a0, prelu_a1, prelu_a2,
           fc_w, fc_b, attn):
    xws = _mm((feat0, feat1, feat2), (W0, W1, W2))

    aggps, degps = [], []
    for xw, ei in zip(xws, (edge_index0, edge_index1, edge_index2)):
        srcp, dstp = _pad_edges(ei)
        aggp, degp = _get_sc_agg()(xw, srcp, dstp)
        aggps.append(aggp)
        degps.append(degp.reshape(NW, NPAD)[:, :N].transpose(1, 0))

    zrow = jnp.zeros((8, D), jnp.float32)
    bias_pad = zrow.at[0].set(b0).at[1].set(b1).at[2].set(b2)
    apad = zrow.at[0].set(prelu_a0).at[1].set(prelu_a1).at[2].set(prelu_a2)
    fcb_pad = zrow.at[0].set(fc_b)
    attn_pad = zrow.at[0].set(attn[0])

    h0, h1, h2, ssum = _norm(aggps, degps, fc_w.T, fcb_pad, bias_pad, apad)
    return _comb((h0, h1, h2), ssum, attn_pad)
